# baseline (speedup 1.0000x reference)
import functools

import jax
import jax.numpy as jnp
from jax import lax
from jax.experimental import pallas as pl
from jax.experimental.pallas import tpu as pltpu
from jax.experimental.pallas import tpu_sc as plsc

N_NODES = 10000
D = 128
N_EDGES = 320000

NC = 2
NS = 16
NW = NC * NS

NPAD = 10240
CHUNK = 128
E2 = 2 * N_EDGES
CPW = -(-E2 // (NW * CHUNK * 8)) * 8
E2P = NW * CPW * CHUNK
PADROW = N_NODES

RPS = NPAD // NS
ZR = 64

_mesh = plsc.VectorSubcoreMesh(
    core_axis_name="c", subcore_axis_name="s", num_cores=NC, num_subcores=NS)

_f32 = jnp.float32


DW = 128


def _deg_body(didx_hbm, degp_hbm, didx_v, ones_v, acc):
    cid = lax.axis_index("c")
    sid = lax.axis_index("s")
    wid = cid * NS + sid

    pltpu.sync_copy(didx_hbm.at[pl.ds(wid * CPW, CPW)], didx_v)

    @pl.loop(0, CHUNK)
    def _zfill(i):
        for k in range(DW // 16):
            ones_v[i, pl.ds(k * 16, 16)] = jnp.zeros((16,), _f32)

    @pl.loop(0, RPS // CHUNK)
    def _zero(k):
        pltpu.sync_copy(ones_v, acc.at[pl.ds(sid * RPS + k * CHUNK, CHUNK)])

    @pl.loop(0, CHUNK)
    def _fill(i):
        ones_v[i, pl.ds(0, 16)] = jnp.ones((16,), _f32)

    plsc.subcore_barrier()

    @pl.loop(0, CPW)
    def _scat(j):
        pltpu.sync_copy(ones_v, acc.at[didx_v.at[j]], add=True)

    plsc.subcore_barrier()
    pltpu.sync_copy(acc.at[pl.ds(sid * RPS, RPS)],
                    degp_hbm.at[cid, pl.ds(sid * RPS, RPS)])


_deg_pass = functools.partial(
    pl.kernel,
    out_type=jax.ShapeDtypeStruct((NC, NPAD, DW), _f32),
    mesh=_mesh,
    scratch_types=[
        pltpu.VMEM((CPW, CHUNK), jnp.int32),
        pltpu.VMEM((CHUNK, DW), _f32),
        pltpu.VMEM_SHARED((NPAD, DW), _f32),
    ],
)(_deg_body)


GRP = 16


C0 = 80
C1 = 2 * CPW - C0


def _agg_body(xs_hbm, sidx_hbm, didx_hbm, agg_hbm,
              sidx_v, didx_v, gbuf0, gbuf1, acc, sem0, sem1):
    cid = lax.axis_index("c")
    sid = lax.axis_index("s")

    @pl.loop(0, CHUNK)
    def _zfill(i):
        for k in range(8):
            gbuf0[i, pl.ds(k * 16, 16)] = jnp.zeros((16,), _f32)

    @pl.loop(0, RPS // CHUNK)
    def _zero(k):
        pltpu.sync_copy(gbuf0, acc.at[pl.ds(sid * RPS + k * CHUNK, CHUNK)])

    plsc.subcore_barrier()

    ngrp = jnp.where(cid == 0, C0 // GRP, C1 // GRP)
    base = jnp.where(cid == 0, sid * C0, NS * C0 + sid * C1)

    @pl.loop(0, ngrp)
    def _grp(g):
        pltpu.sync_copy(sidx_hbm.at[pl.ds(base + g * GRP, GRP)], sidx_v)
        pltpu.sync_copy(didx_hbm.at[pl.ds(base + g * GRP, GRP)], didx_v)
        pltpu.async_copy(xs_hbm.at[sidx_v.at[0]], gbuf0, sem0)
        pltpu.async_copy(xs_hbm.at[sidx_v.at[1]], gbuf1, sem1)

        @pl.loop(0, GRP // 2)
        def _pair(p):
            j0 = 2 * p
            for (j, buf, sem) in ((j0, gbuf0, sem0), (j0 + 1, gbuf1, sem1)):
                pltpu.make_async_copy(xs_hbm.at[sidx_v.at[j]], buf, sem).wait()
                pltpu.sync_copy(buf, acc.at[didx_v.at[j]], add=True)

                @pl.when(j + 2 < GRP)
                def _pre():
                    pltpu.async_copy(xs_hbm.at[sidx_v.at[j + 2]], buf, sem)

    plsc.subcore_barrier()
    pltpu.sync_copy(acc.at[pl.ds(sid * RPS, RPS)],
                    agg_hbm.at[cid, pl.ds(sid * RPS, RPS)])


_agg_pass = functools.partial(
    pl.kernel,
    out_type=jax.ShapeDtypeStruct((NC, NPAD, D), _f32),
    mesh=_mesh,
    scratch_types=[
        pltpu.VMEM((GRP, CHUNK), jnp.int32),
        pltpu.VMEM((GRP, CHUNK), jnp.int32),
        pltpu.VMEM((CHUNK, D), _f32),
        pltpu.VMEM((CHUNK, D), _f32),
        pltpu.VMEM_SHARED((NPAD, D), _f32),
        pltpu.SemaphoreType.DMA,
        pltpu.SemaphoreType.DMA,
    ],
)(_agg_body)


BR = 512


def _tc_a_kernel(x_ref, w_ref, degp_ref, xs_ref, norm_ref):
    deg = degp_ref[0, :, 0:1] + degp_ref[1, :, 0:1] + 1.0
    norm = lax.rsqrt(deg)
    xw = lax.dot_general(x_ref[...], w_ref[...], (((1,), (0,)), ((), ())),
                         precision=lax.Precision.HIGHEST,
                         preferred_element_type=_f32)
    xs_ref[...] = xw * norm
    norm_ref[...] = jnp.broadcast_to(norm, (BR, D))


def _tc_b_kernel(agg_ref, xs1_ref, norm_ref, w_ref, xs2_ref):
    acc = agg_ref[0] + agg_ref[1] + xs1_ref[...]
    h = jnp.maximum(acc * norm_ref[...], 0.0)
    hw = lax.dot_general(h, w_ref[...], (((1,), (0,)), ((), ())),
                         precision=lax.Precision.HIGHEST,
                         preferred_element_type=_f32)
    xs2_ref[...] = hw * norm_ref[...]


def _tc_c_kernel(agg_ref, xs2_ref, norm_ref, o_ref):
    o_ref[...] = (agg_ref[0] + agg_ref[1] + xs2_ref[...]) * norm_ref[...]


def _row_spec(shape3=False):
    if shape3:
        return pl.BlockSpec((NC, BR, D), lambda i: (0, i, 0))
    return pl.BlockSpec((BR, D), lambda i: (i, 0))


_W_SPEC = pl.BlockSpec((D, D), lambda i: (0, 0))
_GRID = (NPAD // BR,)

_tc_a = pl.pallas_call(
    _tc_a_kernel,
    grid=_GRID,
    in_specs=[_row_spec(), _W_SPEC, pl.BlockSpec((NC, BR, DW), lambda i: (0, i, 0))],
    out_specs=[_row_spec(), _row_spec()],
    out_shape=[jax.ShapeDtypeStruct((NPAD, D), _f32),
               jax.ShapeDtypeStruct((NPAD, D), _f32)],
)

_tc_b = pl.pallas_call(
    _tc_b_kernel,
    grid=_GRID,
    in_specs=[_row_spec(True), _row_spec(), _row_spec(), _W_SPEC],
    out_specs=_row_spec(),
    out_shape=jax.ShapeDtypeStruct((NPAD, D), _f32),
)

_tc_c = pl.pallas_call(
    _tc_c_kernel,
    grid=_GRID,
    in_specs=[_row_spec(True), _row_spec(), _row_spec()],
    out_specs=_row_spec(),
    out_shape=jax.ShapeDtypeStruct((NPAD, D), _f32),
)


def kernel(edge_index, emb_weight, W1, W2):
    src = edge_index[0]
    dst = edge_index[1]
    pad = jnp.full((E2P - E2,), PADROW, jnp.int32)
    ids_src = jnp.concatenate([src, dst, pad]).reshape(NW * CPW, CHUNK)
    ids_dst = jnp.concatenate([dst, src, pad]).reshape(NW * CPW, CHUNK)

    x = jnp.pad(emb_weight, ((0, NPAD - N_NODES), (0, 0)))

    degp = _deg_pass(ids_dst)
    xs1, norm = _tc_a(x, W1, degp)
    agg1 = _agg_pass(xs1, ids_src, ids_dst)
    xs2 = _tc_b(agg1, xs1, norm, W2)
    agg2 = _agg_pass(xs2, ids_src, ids_dst)
    out = _tc_c(agg2, xs2, norm)
    return out[:N_NODES]


# speedup vs baseline: 1.1320x; 1.1320x over previous
import functools

import jax
import jax.numpy as jnp
from jax import lax
from jax.experimental import pallas as pl
from jax.experimental.pallas import tpu as pltpu
from jax.experimental.pallas import tpu_sc as plsc

N_NODES = 10000
D = 128
N_EDGES = 320000

NC = 2
NS = 16
NW = NC * NS

NPAD = 10240
CHUNK = 128
E2 = 2 * N_EDGES
CPW = -(-E2 // (NW * CHUNK * 8)) * 8
E2P = NW * CPW * CHUNK
PADROW = N_NODES

RPS = NPAD // NS
ZR = 64

_mesh = plsc.VectorSubcoreMesh(
    core_axis_name="c", subcore_axis_name="s", num_cores=NC, num_subcores=NS)

_f32 = jnp.float32


DW = 128


def _deg_body(didx_hbm, degp_hbm, didx_v, ones_v, acc):
    cid = lax.axis_index("c")
    sid = lax.axis_index("s")
    wid = cid * NS + sid

    pltpu.sync_copy(didx_hbm.at[pl.ds(wid * CPW, CPW)], didx_v)

    @pl.loop(0, CHUNK)
    def _zfill(i):
        for k in range(DW // 16):
            ones_v[i, pl.ds(k * 16, 16)] = jnp.zeros((16,), _f32)

    @pl.loop(0, RPS // CHUNK)
    def _zero(k):
        pltpu.sync_copy(ones_v, acc.at[pl.ds(sid * RPS + k * CHUNK, CHUNK)])

    @pl.loop(0, CHUNK)
    def _fill(i):
        ones_v[i, pl.ds(0, 16)] = jnp.ones((16,), _f32)

    plsc.subcore_barrier()

    @pl.loop(0, CPW)
    def _scat(j):
        pltpu.sync_copy(ones_v, acc.at[didx_v.at[j]], add=True)

    plsc.subcore_barrier()
    pltpu.sync_copy(acc.at[pl.ds(sid * RPS, RPS)],
                    degp_hbm.at[cid, pl.ds(sid * RPS, RPS)])


_deg_pass = functools.partial(
    pl.kernel,
    out_type=jax.ShapeDtypeStruct((NC, NPAD, DW), _f32),
    mesh=_mesh,
    scratch_types=[
        pltpu.VMEM((CPW, CHUNK), jnp.int32),
        pltpu.VMEM((CHUNK, DW), _f32),
        pltpu.VMEM_SHARED((NPAD, DW), _f32),
    ],
)(_deg_body)


GRP = 16


C0 = 240
C1 = 2 * CPW - C0

Q = 4
SUB = CHUNK // Q


def _agg_body(xs_hbm, sidx_hbm, didx_hbm, agg_hbm,
              sidx_v, didx_v, gbuf0, gbuf1, acc, sem0, sem1):
    cid = lax.axis_index("c")
    sid = lax.axis_index("s")

    @pl.loop(0, CHUNK)
    def _zfill(i):
        for k in range(8):
            gbuf0[i, pl.ds(k * 16, 16)] = jnp.zeros((16,), _f32)

    @pl.loop(0, RPS // CHUNK)
    def _zero(k):
        pltpu.sync_copy(gbuf0, acc.at[pl.ds(sid * RPS + k * CHUNK, CHUNK)])

    plsc.subcore_barrier()

    ngrp = jnp.where(cid == 0, C0 // GRP, C1 // GRP)
    base = jnp.where(cid == 0, sid * C0, NS * C0 + sid * C1)

    @pl.loop(0, ngrp)
    def _grp(g):
        pltpu.sync_copy(sidx_hbm.at[pl.ds(base + g * GRP, GRP)], sidx_v)
        pltpu.sync_copy(didx_hbm.at[pl.ds(base + g * GRP, GRP)], didx_v)
        def _fire(j, buf, sem):
            for q in range(Q):
                pltpu.async_copy(
                    xs_hbm.at[sidx_v.at[j, pl.ds(q * SUB, SUB)]],
                    buf.at[pl.ds(q * SUB, SUB)], sem)

        def _drain(j, buf, sem):
            for q in range(Q):
                pltpu.make_async_copy(
                    xs_hbm.at[sidx_v.at[j, pl.ds(q * SUB, SUB)]],
                    buf.at[pl.ds(q * SUB, SUB)], sem).wait()

        _fire(0, gbuf0, sem0)
        _fire(1, gbuf1, sem1)

        @pl.loop(0, GRP // 2)
        def _pair(p):
            j0 = 2 * p
            for (j, buf, sem) in ((j0, gbuf0, sem0), (j0 + 1, gbuf1, sem1)):
                _drain(j, buf, sem)
                pltpu.sync_copy(buf, acc.at[didx_v.at[j]], add=True)

                @pl.when(j + 2 < GRP)
                def _pre():
                    _fire(j + 2, buf, sem)

    plsc.subcore_barrier()
    pltpu.sync_copy(acc.at[pl.ds(sid * RPS, RPS)],
                    agg_hbm.at[cid, pl.ds(sid * RPS, RPS)])


_agg_pass = functools.partial(
    pl.kernel,
    out_type=jax.ShapeDtypeStruct((NC, NPAD, D), _f32),
    mesh=_mesh,
    scratch_types=[
        pltpu.VMEM((GRP, CHUNK), jnp.int32),
        pltpu.VMEM((GRP, CHUNK), jnp.int32),
        pltpu.VMEM((CHUNK, D), _f32),
        pltpu.VMEM((CHUNK, D), _f32),
        pltpu.VMEM_SHARED((NPAD, D), _f32),
        pltpu.SemaphoreType.DMA,
        pltpu.SemaphoreType.DMA,
    ],
)(_agg_body)


BR = 512


def _tc_a_kernel(x_ref, w_ref, degp_ref, xs_ref, norm_ref):
    deg = degp_ref[0, :, 0:1] + degp_ref[1, :, 0:1] + 1.0
    norm = lax.rsqrt(deg)
    xw = lax.dot_general(x_ref[...], w_ref[...], (((1,), (0,)), ((), ())),
                         precision=lax.Precision.HIGHEST,
                         preferred_element_type=_f32)
    xs_ref[...] = xw * norm
    norm_ref[...] = jnp.broadcast_to(norm, (BR, D))


def _tc_b_kernel(agg_ref, xs1_ref, norm_ref, w_ref, xs2_ref):
    acc = agg_ref[0] + agg_ref[1] + xs1_ref[...]
    h = jnp.maximum(acc * norm_ref[...], 0.0)
    hw = lax.dot_general(h, w_ref[...], (((1,), (0,)), ((), ())),
                         precision=lax.Precision.HIGHEST,
                         preferred_element_type=_f32)
    xs2_ref[...] = hw * norm_ref[...]


def _tc_c_kernel(agg_ref, xs2_ref, norm_ref, o_ref):
    o_ref[...] = (agg_ref[0] + agg_ref[1] + xs2_ref[...]) * norm_ref[...]


def _row_spec(shape3=False):
    if shape3:
        return pl.BlockSpec((NC, BR, D), lambda i: (0, i, 0))
    return pl.BlockSpec((BR, D), lambda i: (i, 0))


_W_SPEC = pl.BlockSpec((D, D), lambda i: (0, 0))
_GRID = (NPAD // BR,)

_tc_a = pl.pallas_call(
    _tc_a_kernel,
    grid=_GRID,
    in_specs=[_row_spec(), _W_SPEC, pl.BlockSpec((NC, BR, DW), lambda i: (0, i, 0))],
    out_specs=[_row_spec(), _row_spec()],
    out_shape=[jax.ShapeDtypeStruct((NPAD, D), _f32),
               jax.ShapeDtypeStruct((NPAD, D), _f32)],
)

_tc_b = pl.pallas_call(
    _tc_b_kernel,
    grid=_GRID,
    in_specs=[_row_spec(True), _row_spec(), _row_spec(), _W_SPEC],
    out_specs=_row_spec(),
    out_shape=jax.ShapeDtypeStruct((NPAD, D), _f32),
)

_tc_c = pl.pallas_call(
    _tc_c_kernel,
    grid=_GRID,
    in_specs=[_row_spec(True), _row_spec(), _row_spec()],
    out_specs=_row_spec(),
    out_shape=jax.ShapeDtypeStruct((NPAD, D), _f32),
)


def kernel(edge_index, emb_weight, W1, W2):
    src = edge_index[0]
    dst = edge_index[1]
    pad = jnp.full((E2P - E2,), PADROW, jnp.int32)
    ids_src = jnp.concatenate([src, dst, pad]).reshape(NW * CPW, CHUNK)
    ids_dst = jnp.concatenate([dst, src, pad]).reshape(NW * CPW, CHUNK)

    x = jnp.pad(emb_weight, ((0, NPAD - N_NODES), (0, 0)))

    degp = _deg_pass(ids_dst)
    xs1, norm = _tc_a(x, W1, degp)
    agg1 = _agg_pass(xs1, ids_src, ids_dst)
    xs2 = _tc_b(agg1, xs1, norm, W2)
    agg2 = _agg_pass(xs2, ids_src, ids_dst)
    out = _tc_c(agg2, xs2, norm)
    return out[:N_NODES]


# speedup vs baseline: 3.1508x; 2.7834x over previous
import functools

import jax
import jax.numpy as jnp
from jax import lax
from jax.experimental import pallas as pl
from jax.experimental.pallas import tpu as pltpu
from jax.experimental.pallas import tpu_sc as plsc

N_NODES = 10000
D = 128
N_EDGES = 320000

NC = 2
NS = 16
NW = NC * NS

NPAD = 10240
CHUNK = 128
E2 = 2 * N_EDGES
CPW = -(-E2 // (NW * CHUNK * 8)) * 8
E2P = NW * CPW * CHUNK
PADROW = N_NODES

RPS = NPAD // NS
ZR = 64

_mesh = plsc.VectorSubcoreMesh(
    core_axis_name="c", subcore_axis_name="s", num_cores=NC, num_subcores=NS)

_f32 = jnp.float32


DW = 128


def _deg_body(didx_hbm, degp_hbm, didx_v, ones_v, acc):
    cid = lax.axis_index("c")
    sid = lax.axis_index("s")
    wid = cid * NS + sid

    pltpu.sync_copy(didx_hbm.at[pl.ds(wid * CPW, CPW)], didx_v)

    @pl.loop(0, CHUNK)
    def _zfill(i):
        for k in range(DW // 16):
            ones_v[i, pl.ds(k * 16, 16)] = jnp.zeros((16,), _f32)

    @pl.loop(0, RPS // CHUNK)
    def _zero(k):
        pltpu.sync_copy(ones_v, acc.at[pl.ds(sid * RPS + k * CHUNK, CHUNK)])

    @pl.loop(0, CHUNK)
    def _fill(i):
        ones_v[i, pl.ds(0, 16)] = jnp.ones((16,), _f32)

    plsc.subcore_barrier()

    @pl.loop(0, CPW)
    def _scat(j):
        pltpu.sync_copy(ones_v, acc.at[didx_v.at[j]], add=True)

    plsc.subcore_barrier()
    pltpu.sync_copy(acc.at[pl.ds(sid * RPS, RPS)],
                    degp_hbm.at[cid, pl.ds(sid * RPS, RPS)])


_deg_pass = functools.partial(
    pl.kernel,
    out_type=jax.ShapeDtypeStruct((NC, NPAD, DW), _f32),
    mesh=_mesh,
    scratch_types=[
        pltpu.VMEM((CPW, CHUNK), jnp.int32),
        pltpu.VMEM((CHUNK, DW), _f32),
        pltpu.VMEM_SHARED((NPAD, DW), _f32),
    ],
)(_deg_body)


GRP = 16


C0 = 160
C1 = 2 * CPW - C0

Q = 4
SUB = CHUNK // Q


def _agg_body(xs_hbm, sidx_hbm, didx_hbm, agg_hbm,
              sidx_v, didx_v, gbuf0, gbuf1, acc, sem0, sem1):
    cid = lax.axis_index("c")
    sid = lax.axis_index("s")

    @pl.loop(0, CHUNK)
    def _zfill(i):
        for k in range(8):
            gbuf0[i, pl.ds(k * 16, 16)] = jnp.zeros((16,), _f32)

    @pl.loop(0, RPS // CHUNK)
    def _zero(k):
        pltpu.sync_copy(gbuf0, acc.at[pl.ds(sid * RPS + k * CHUNK, CHUNK)])

    plsc.subcore_barrier()

    ngrp = jnp.where(cid == 0, C0 // GRP, C1 // GRP)
    base = jnp.where(cid == 0, sid * C0, NS * C0 + sid * C1)

    @pl.loop(0, ngrp)
    def _grp(g):
        pltpu.sync_copy(sidx_hbm.at[pl.ds(base + g * GRP, GRP)], sidx_v)
        pltpu.sync_copy(didx_hbm.at[pl.ds(base + g * GRP, GRP)], didx_v)
        def _fire(j, buf, sem):
            for q in range(Q):
                pltpu.async_copy(
                    xs_hbm.at[sidx_v.at[j, pl.ds(q * SUB, SUB)]],
                    buf.at[pl.ds(q * SUB, SUB)], sem)

        def _drain(j, buf, sem):
            for q in range(Q):
                pltpu.make_async_copy(
                    xs_hbm.at[sidx_v.at[j, pl.ds(q * SUB, SUB)]],
                    buf.at[pl.ds(q * SUB, SUB)], sem).wait()

        _fire(0, gbuf0, sem0)
        _fire(1, gbuf1, sem1)

        @pl.loop(0, GRP // 2)
        def _pair(p):
            j0 = 2 * p
            for (j, buf, sem) in ((j0, gbuf0, sem0), (j0 + 1, gbuf1, sem1)):
                _drain(j, buf, sem)
                pltpu.sync_copy(buf, acc.at[didx_v.at[j]], add=True)

                @pl.when(j + 2 < GRP)
                def _pre():
                    _fire(j + 2, buf, sem)

    plsc.subcore_barrier()
    pltpu.sync_copy(acc.at[pl.ds(sid * RPS, RPS)],
                    agg_hbm.at[cid, pl.ds(sid * RPS, RPS)])


_agg_pass = functools.partial(
    pl.kernel,
    out_type=jax.ShapeDtypeStruct((NC, NPAD, D), _f32),
    mesh=_mesh,
    scratch_types=[
        pltpu.VMEM((GRP, CHUNK), jnp.int32),
        pltpu.VMEM((GRP, CHUNK), jnp.int32),
        pltpu.VMEM((CHUNK, D), _f32),
        pltpu.VMEM((CHUNK, D), _f32),
        pltpu.VMEM_SHARED((NPAD, D), _f32),
        pltpu.SemaphoreType.DMA,
        pltpu.SemaphoreType.DMA,
    ],
)(_agg_body)


BR = 512


def _tc_a_kernel(x_ref, w_ref, degp_ref, xs_ref, norm_ref):
    deg = degp_ref[0, :, 0:1] + degp_ref[1, :, 0:1] + 1.0
    norm = lax.rsqrt(deg)
    xw = lax.dot_general(x_ref[...], w_ref[...], (((1,), (0,)), ((), ())),
                         precision=lax.Precision.HIGHEST,
                         preferred_element_type=_f32)
    xs_ref[...] = xw * norm
    norm_ref[...] = jnp.broadcast_to(norm, (BR, D))


def _tc_b_kernel(agg_ref, xs1_ref, norm_ref, w_ref, xs2_ref):
    acc = agg_ref[0] + agg_ref[1] + xs1_ref[...]
    h = jnp.maximum(acc * norm_ref[...], 0.0)
    hw = lax.dot_general(h, w_ref[...], (((1,), (0,)), ((), ())),
                         precision=lax.Precision.HIGHEST,
                         preferred_element_type=_f32)
    xs2_ref[...] = hw * norm_ref[...]


def _tc_c_kernel(agg_ref, xs2_ref, norm_ref, o_ref):
    o_ref[...] = (agg_ref[0] + agg_ref[1] + xs2_ref[...]) * norm_ref[...]


def _row_spec(shape3=False):
    if shape3:
        return pl.BlockSpec((NC, BR, D), lambda i: (0, i, 0))
    return pl.BlockSpec((BR, D), lambda i: (i, 0))


_W_SPEC = pl.BlockSpec((D, D), lambda i: (0, 0))
_GRID = (NPAD // BR,)

_tc_a = pl.pallas_call(
    _tc_a_kernel,
    grid=_GRID,
    in_specs=[_row_spec(), _W_SPEC, pl.BlockSpec((NC, BR, DW), lambda i: (0, i, 0))],
    out_specs=[_row_spec(), _row_spec()],
    out_shape=[jax.ShapeDtypeStruct((NPAD, D), _f32),
               jax.ShapeDtypeStruct((NPAD, D), _f32)],
)

_tc_b = pl.pallas_call(
    _tc_b_kernel,
    grid=_GRID,
    in_specs=[_row_spec(True), _row_spec(), _row_spec(), _W_SPEC],
    out_specs=_row_spec(),
    out_shape=jax.ShapeDtypeStruct((NPAD, D), _f32),
)

_tc_c = pl.pallas_call(
    _tc_c_kernel,
    grid=_GRID,
    in_specs=[_row_spec(True), _row_spec(), _row_spec()],
    out_specs=_row_spec(),
    out_shape=jax.ShapeDtypeStruct((NPAD, D), _f32),
)


def kernel(edge_index, emb_weight, W1, W2):
    src = edge_index[0]
    dst = edge_index[1]
    pad = PADROW + (jnp.arange(E2P - E2, dtype=jnp.int32) % (NPAD - N_NODES))
    ids_src = jnp.concatenate([src, dst, pad]).reshape(NW * CPW, CHUNK)
    ids_dst = jnp.concatenate([dst, src, pad]).reshape(NW * CPW, CHUNK)

    x = jnp.pad(emb_weight, ((0, NPAD - N_NODES), (0, 0)))

    degp = _deg_pass(ids_dst)
    xs1, norm = _tc_a(x, W1, degp)
    agg1 = _agg_pass(xs1, ids_src, ids_dst)
    xs2 = _tc_b(agg1, xs1, norm, W2)
    agg2 = _agg_pass(xs2, ids_src, ids_dst)
    out = _tc_c(agg2, xs2, norm)
    return out[:N_NODES]


# speedup vs baseline: 3.3084x; 1.0500x over previous
import functools

import jax
import jax.numpy as jnp
from jax import lax
from jax.experimental import pallas as pl
from jax.experimental.pallas import tpu as pltpu
from jax.experimental.pallas import tpu_sc as plsc

N_NODES = 10000
D = 128
N_EDGES = 320000

NC = 2
NS = 16
NW = NC * NS

NPAD = 10240
CHUNK = 128
E2 = 2 * N_EDGES
CPW = -(-E2 // (NW * CHUNK * 8)) * 8
E2P = NW * CPW * CHUNK
PADROW = N_NODES

RPS = NPAD // NS
ZR = 64

_mesh = plsc.VectorSubcoreMesh(
    core_axis_name="c", subcore_axis_name="s", num_cores=NC, num_subcores=NS)

_f32 = jnp.float32


DW = 128


def _deg_body(didx_hbm, degp_hbm, didx_v, ones_v, acc):
    cid = lax.axis_index("c")
    sid = lax.axis_index("s")
    wid = cid * NS + sid

    pltpu.sync_copy(didx_hbm.at[pl.ds(wid * CPW, CPW)], didx_v)

    @pl.loop(0, CHUNK)
    def _zfill(i):
        for k in range(DW // 16):
            ones_v[i, pl.ds(k * 16, 16)] = jnp.zeros((16,), _f32)

    @pl.loop(0, RPS // CHUNK)
    def _zero(k):
        pltpu.sync_copy(ones_v, acc.at[pl.ds(sid * RPS + k * CHUNK, CHUNK)])

    @pl.loop(0, CHUNK)
    def _fill(i):
        ones_v[i, pl.ds(0, 16)] = jnp.ones((16,), _f32)

    plsc.subcore_barrier()

    @pl.loop(0, CPW)
    def _scat(j):
        pltpu.sync_copy(ones_v, acc.at[didx_v.at[j]], add=True)

    plsc.subcore_barrier()
    pltpu.sync_copy(acc.at[pl.ds(sid * RPS, RPS)],
                    degp_hbm.at[cid, pl.ds(sid * RPS, RPS)])


_deg_pass = functools.partial(
    pl.kernel,
    out_type=jax.ShapeDtypeStruct((NC, NPAD, DW), _f32),
    mesh=_mesh,
    scratch_types=[
        pltpu.VMEM((CPW, CHUNK), jnp.int32),
        pltpu.VMEM((CHUNK, DW), _f32),
        pltpu.VMEM_SHARED((NPAD, DW), _f32),
    ],
)(_deg_body)


GRP = 40


C0 = 160
C1 = 2 * CPW - C0

Q = 4
SUB = CHUNK // Q


def _agg_body(xs_hbm, sidx_hbm, didx_hbm, agg_hbm,
              sidx_v, didx_v, gbuf0, gbuf1, acc, sem0, sem1):
    cid = lax.axis_index("c")
    sid = lax.axis_index("s")

    @pl.loop(0, CHUNK)
    def _zfill(i):
        for k in range(8):
            gbuf0[i, pl.ds(k * 16, 16)] = jnp.zeros((16,), _f32)

    @pl.loop(0, RPS // CHUNK)
    def _zero(k):
        pltpu.sync_copy(gbuf0, acc.at[pl.ds(sid * RPS + k * CHUNK, CHUNK)])

    plsc.subcore_barrier()

    ngrp = jnp.where(cid == 0, C0 // GRP, C1 // GRP)
    base = jnp.where(cid == 0, sid * C0, NS * C0 + sid * C1)

    @pl.loop(0, ngrp)
    def _grp(g):
        pltpu.sync_copy(sidx_hbm.at[pl.ds(base + g * GRP, GRP)], sidx_v)
        pltpu.sync_copy(didx_hbm.at[pl.ds(base + g * GRP, GRP)], didx_v)
        def _fire(j, buf, sem):
            for q in range(Q):
                pltpu.async_copy(
                    xs_hbm.at[sidx_v.at[j, pl.ds(q * SUB, SUB)]],
                    buf.at[pl.ds(q * SUB, SUB)], sem)

        def _drain(j, buf, sem):
            for q in range(Q):
                pltpu.make_async_copy(
                    xs_hbm.at[sidx_v.at[j, pl.ds(q * SUB, SUB)]],
                    buf.at[pl.ds(q * SUB, SUB)], sem).wait()

        _fire(0, gbuf0, sem0)
        _fire(1, gbuf1, sem1)

        @pl.loop(0, GRP // 2)
        def _pair(p):
            j0 = 2 * p
            for (j, buf, sem) in ((j0, gbuf0, sem0), (j0 + 1, gbuf1, sem1)):
                _drain(j, buf, sem)
                pltpu.sync_copy(buf, acc.at[didx_v.at[j]], add=True)

                @pl.when(j + 2 < GRP)
                def _pre():
                    _fire(j + 2, buf, sem)

    plsc.subcore_barrier()
    pltpu.sync_copy(acc.at[pl.ds(sid * RPS, RPS)],
                    agg_hbm.at[cid, pl.ds(sid * RPS, RPS)])


_agg_pass = functools.partial(
    pl.kernel,
    out_type=jax.ShapeDtypeStruct((NC, NPAD, D), _f32),
    mesh=_mesh,
    scratch_types=[
        pltpu.VMEM((GRP, CHUNK), jnp.int32),
        pltpu.VMEM((GRP, CHUNK), jnp.int32),
        pltpu.VMEM((CHUNK, D), _f32),
        pltpu.VMEM((CHUNK, D), _f32),
        pltpu.VMEM_SHARED((NPAD, D), _f32),
        pltpu.SemaphoreType.DMA,
        pltpu.SemaphoreType.DMA,
    ],
)(_agg_body)


BR = 512


def _tc_a_kernel(x_ref, w_ref, degp_ref, xs_ref, norm_ref):
    deg = degp_ref[0, :, 0:1] + degp_ref[1, :, 0:1] + 1.0
    norm = lax.rsqrt(deg)
    xw = lax.dot_general(x_ref[...], w_ref[...], (((1,), (0,)), ((), ())),
                         precision=lax.Precision.HIGHEST,
                         preferred_element_type=_f32)
    xs_ref[...] = xw * norm
    norm_ref[...] = jnp.broadcast_to(norm, (BR, D))


def _tc_b_kernel(agg_ref, xs1_ref, norm_ref, w_ref, xs2_ref):
    acc = agg_ref[0] + agg_ref[1] + xs1_ref[...]
    h = jnp.maximum(acc * norm_ref[...], 0.0)
    hw = lax.dot_general(h, w_ref[...], (((1,), (0,)), ((), ())),
                         precision=lax.Precision.HIGHEST,
                         preferred_element_type=_f32)
    xs2_ref[...] = hw * norm_ref[...]


def _tc_c_kernel(agg_ref, xs2_ref, norm_ref, o_ref):
    o_ref[...] = (agg_ref[0] + agg_ref[1] + xs2_ref[...]) * norm_ref[...]


def _row_spec(shape3=False):
    if shape3:
        return pl.BlockSpec((NC, BR, D), lambda i: (0, i, 0))
    return pl.BlockSpec((BR, D), lambda i: (i, 0))


_W_SPEC = pl.BlockSpec((D, D), lambda i: (0, 0))
_GRID = (NPAD // BR,)

_tc_a = pl.pallas_call(
    _tc_a_kernel,
    grid=_GRID,
    in_specs=[_row_spec(), _W_SPEC, pl.BlockSpec((NC, BR, DW), lambda i: (0, i, 0))],
    out_specs=[_row_spec(), _row_spec()],
    out_shape=[jax.ShapeDtypeStruct((NPAD, D), _f32),
               jax.ShapeDtypeStruct((NPAD, D), _f32)],
)

_tc_b = pl.pallas_call(
    _tc_b_kernel,
    grid=_GRID,
    in_specs=[_row_spec(True), _row_spec(), _row_spec(), _W_SPEC],
    out_specs=_row_spec(),
    out_shape=jax.ShapeDtypeStruct((NPAD, D), _f32),
)

_tc_c = pl.pallas_call(
    _tc_c_kernel,
    grid=_GRID,
    in_specs=[_row_spec(True), _row_spec(), _row_spec()],
    out_specs=_row_spec(),
    out_shape=jax.ShapeDtypeStruct((NPAD, D), _f32),
)


def kernel(edge_index, emb_weight, W1, W2):
    src = edge_index[0]
    dst = edge_index[1]
    pad = PADROW + (jnp.arange(E2P - E2, dtype=jnp.int32) % (NPAD - N_NODES))
    ids_src = jnp.concatenate([src, dst, pad]).reshape(NW * CPW, CHUNK)
    ids_dst = jnp.concatenate([dst, src, pad]).reshape(NW * CPW, CHUNK)

    x = jnp.pad(emb_weight, ((0, NPAD - N_NODES), (0, 0)))

    degp = _deg_pass(ids_dst)
    xs1, norm = _tc_a(x, W1, degp)
    agg1 = _agg_pass(xs1, ids_src, ids_dst)
    xs2 = _tc_b(agg1, xs1, norm, W2)
    agg2 = _agg_pass(xs2, ids_src, ids_dst)
    out = _tc_c(agg2, xs2, norm)
    return out[:N_NODES]


# speedup vs baseline: 3.3491x; 1.0123x over previous
import functools

import jax
import jax.numpy as jnp
from jax import lax
from jax.experimental import pallas as pl
from jax.experimental.pallas import tpu as pltpu
from jax.experimental.pallas import tpu_sc as plsc

N_NODES = 10000
D = 128
N_EDGES = 320000

NC = 2
NS = 16
NW = NC * NS

NPAD = 10240
CHUNK = 128
E2 = 2 * N_EDGES
CPW = -(-E2 // (NW * CHUNK * 8)) * 8
E2P = NW * CPW * CHUNK
PADROW = N_NODES

RPS = NPAD // NS
ZR = 64

_mesh = plsc.VectorSubcoreMesh(
    core_axis_name="c", subcore_axis_name="s", num_cores=NC, num_subcores=NS)

_f32 = jnp.float32


DW = 128


def _deg_body(didx_hbm, degp_hbm, didx_v, ones_v, acc):
    cid = lax.axis_index("c")
    sid = lax.axis_index("s")
    wid = cid * NS + sid

    pltpu.sync_copy(didx_hbm.at[pl.ds(wid * CPW, CPW)], didx_v)

    @pl.loop(0, CHUNK)
    def _zfill(i):
        for k in range(DW // 16):
            ones_v[i, pl.ds(k * 16, 16)] = jnp.zeros((16,), _f32)

    @pl.loop(0, RPS // CHUNK)
    def _zero(k):
        pltpu.sync_copy(ones_v, acc.at[pl.ds(sid * RPS + k * CHUNK, CHUNK)])

    @pl.loop(0, CHUNK)
    def _fill(i):
        ones_v[i, pl.ds(0, 16)] = jnp.ones((16,), _f32)

    plsc.subcore_barrier()

    @pl.loop(0, CPW)
    def _scat(j):
        pltpu.sync_copy(ones_v, acc.at[didx_v.at[j]], add=True)

    plsc.subcore_barrier()
    pltpu.sync_copy(acc.at[pl.ds(sid * RPS, RPS)],
                    degp_hbm.at[cid, pl.ds(sid * RPS, RPS)])


_deg_pass = functools.partial(
    pl.kernel,
    out_type=jax.ShapeDtypeStruct((NC, NPAD, DW), _f32),
    mesh=_mesh,
    scratch_types=[
        pltpu.VMEM((CPW, CHUNK), jnp.int32),
        pltpu.VMEM((CHUNK, DW), _f32),
        pltpu.VMEM_SHARED((NPAD, DW), _f32),
    ],
)(_deg_body)


GRP = 40

CPH = E2P // (2 * CHUNK)
CPP = CPH // NW

Q = 4
SUB = CHUNK // Q


def _agg_body(xs_hbm, ids_hbm, agg_hbm,
              sidx_v, didx_v, gbuf0, gbuf1, acc, sem0, sem1):
    cid = lax.axis_index("c")
    sid = lax.axis_index("s")

    @pl.loop(0, CHUNK)
    def _zfill(i):
        for k in range(8):
            gbuf0[i, pl.ds(k * 16, 16)] = jnp.zeros((16,), _f32)

    @pl.loop(0, RPS // CHUNK)
    def _zero(k):
        pltpu.sync_copy(gbuf0, acc.at[pl.ds(sid * RPS + k * CHUNK, CHUNK)])

    plsc.subcore_barrier()

    base_c = (cid * NS + sid) * CPP

    for (s_off, d_off) in ((base_c, CPH + base_c), (CPH + base_c, base_c)):

        @pl.loop(0, CPP // GRP)
        def _grp(g):
            pltpu.sync_copy(ids_hbm.at[pl.ds(s_off + g * GRP, GRP)], sidx_v)
            pltpu.sync_copy(ids_hbm.at[pl.ds(d_off + g * GRP, GRP)], didx_v)

            def _fire(j, buf, sem):
                for q in range(Q):
                    pltpu.async_copy(
                        xs_hbm.at[sidx_v.at[j, pl.ds(q * SUB, SUB)]],
                        buf.at[pl.ds(q * SUB, SUB)], sem)

            def _drain(j, buf, sem):
                for q in range(Q):
                    pltpu.make_async_copy(
                        xs_hbm.at[sidx_v.at[j, pl.ds(q * SUB, SUB)]],
                        buf.at[pl.ds(q * SUB, SUB)], sem).wait()

            _fire(0, gbuf0, sem0)
            _fire(1, gbuf1, sem1)

            @pl.loop(0, GRP // 2)
            def _pair(p):
                j0 = 2 * p
                for (j, buf, sem) in ((j0, gbuf0, sem0), (j0 + 1, gbuf1, sem1)):
                    _drain(j, buf, sem)
                    pltpu.sync_copy(buf, acc.at[didx_v.at[j]], add=True)

                    @pl.when(j + 2 < GRP)
                    def _pre():
                        _fire(j + 2, buf, sem)

    plsc.subcore_barrier()
    pltpu.sync_copy(acc.at[pl.ds(sid * RPS, RPS)],
                    agg_hbm.at[cid, pl.ds(sid * RPS, RPS)])


_agg_pass = functools.partial(
    pl.kernel,
    out_type=jax.ShapeDtypeStruct((NC, NPAD, D), _f32),
    mesh=_mesh,
    scratch_types=[
        pltpu.VMEM((GRP, CHUNK), jnp.int32),
        pltpu.VMEM((GRP, CHUNK), jnp.int32),
        pltpu.VMEM((CHUNK, D), _f32),
        pltpu.VMEM((CHUNK, D), _f32),
        pltpu.VMEM_SHARED((NPAD, D), _f32),
        pltpu.SemaphoreType.DMA,
        pltpu.SemaphoreType.DMA,
    ],
)(_agg_body)


BR = 512


def _tc_a_kernel(x_ref, w_ref, degp_ref, xs_ref, norm_ref):
    deg = degp_ref[0, :, 0:1] + degp_ref[1, :, 0:1] + 1.0
    norm = lax.rsqrt(deg)
    xw = lax.dot_general(x_ref[...], w_ref[...], (((1,), (0,)), ((), ())),
                         precision=lax.Precision.HIGHEST,
                         preferred_element_type=_f32)
    xs_ref[...] = xw * norm
    norm_ref[...] = jnp.broadcast_to(norm, (BR, D))


def _tc_b_kernel(agg_ref, xs1_ref, norm_ref, w_ref, xs2_ref):
    acc = agg_ref[0] + agg_ref[1] + xs1_ref[...]
    h = jnp.maximum(acc * norm_ref[...], 0.0)
    hw = lax.dot_general(h, w_ref[...], (((1,), (0,)), ((), ())),
                         precision=lax.Precision.HIGHEST,
                         preferred_element_type=_f32)
    xs2_ref[...] = hw * norm_ref[...]


def _tc_c_kernel(agg_ref, xs2_ref, norm_ref, o_ref):
    o_ref[...] = (agg_ref[0] + agg_ref[1] + xs2_ref[...]) * norm_ref[...]


def _row_spec(shape3=False):
    if shape3:
        return pl.BlockSpec((NC, BR, D), lambda i: (0, i, 0))
    return pl.BlockSpec((BR, D), lambda i: (i, 0))


_W_SPEC = pl.BlockSpec((D, D), lambda i: (0, 0))
_GRID = (NPAD // BR,)

_tc_a = pl.pallas_call(
    _tc_a_kernel,
    grid=_GRID,
    in_specs=[_row_spec(), _W_SPEC, pl.BlockSpec((NC, BR, DW), lambda i: (0, i, 0))],
    out_specs=[_row_spec(), _row_spec()],
    out_shape=[jax.ShapeDtypeStruct((NPAD, D), _f32),
               jax.ShapeDtypeStruct((NPAD, D), _f32)],
)

_tc_b = pl.pallas_call(
    _tc_b_kernel,
    grid=_GRID,
    in_specs=[_row_spec(True), _row_spec(), _row_spec(), _W_SPEC],
    out_specs=_row_spec(),
    out_shape=jax.ShapeDtypeStruct((NPAD, D), _f32),
)

_tc_c = pl.pallas_call(
    _tc_c_kernel,
    grid=_GRID,
    in_specs=[_row_spec(True), _row_spec(), _row_spec()],
    out_specs=_row_spec(),
    out_shape=jax.ShapeDtypeStruct((NPAD, D), _f32),
)


def kernel(edge_index, emb_weight, W1, W2):
    npad_e = CPH * CHUNK - N_EDGES
    pad = PADROW + (jnp.arange(npad_e, dtype=jnp.int32) % (NPAD - N_NODES))
    ei_pad = jnp.concatenate([edge_index, jnp.stack([pad, pad])], axis=1)
    ids = ei_pad.reshape(2 * CPH, CHUNK)

    x = jnp.pad(emb_weight, ((0, NPAD - N_NODES), (0, 0)))

    degp = _deg_pass(ids)
    xs1, norm = _tc_a(x, W1, degp)
    agg1 = _agg_pass(xs1, ids)
    xs2 = _tc_b(agg1, xs1, norm, W2)
    agg2 = _agg_pass(xs2, ids)
    out = _tc_c(agg2, xs2, norm)
    return out[:N_NODES]


# speedup vs baseline: 3.3518x; 1.0008x over previous
import functools

import jax
import jax.numpy as jnp
from jax import lax
from jax.experimental import pallas as pl
from jax.experimental.pallas import tpu as pltpu
from jax.experimental.pallas import tpu_sc as plsc

N_NODES = 10000
D = 128
N_EDGES = 320000

NC = 2
NS = 16
NW = NC * NS

NPAD = 10240
CHUNK = 128
E2 = 2 * N_EDGES
CPW = -(-E2 // (NW * CHUNK * 8)) * 8
E2P = NW * CPW * CHUNK
PADROW = N_NODES

RPS = NPAD // NS
ZR = 64

_mesh = plsc.VectorSubcoreMesh(
    core_axis_name="c", subcore_axis_name="s", num_cores=NC, num_subcores=NS)

_f32 = jnp.float32


DW = 128


def _deg_body(didx_hbm, degp_hbm, didx_v, ones_v, acc, dsem):
    cid = lax.axis_index("c")
    sid = lax.axis_index("s")
    wid = cid * NS + sid

    pltpu.sync_copy(didx_hbm.at[pl.ds(wid * CPW, CPW)], didx_v)

    @pl.loop(0, CHUNK)
    def _zfill(i):
        for k in range(DW // 16):
            ones_v[i, pl.ds(k * 16, 16)] = jnp.zeros((16,), _f32)

    @pl.loop(0, RPS // CHUNK)
    def _zero(k):
        pltpu.sync_copy(ones_v, acc.at[pl.ds(sid * RPS + k * CHUNK, CHUNK)])

    @pl.loop(0, CHUNK)
    def _fill(i):
        ones_v[i, pl.ds(0, 16)] = jnp.ones((16,), _f32)

    plsc.subcore_barrier()

    @pl.loop(0, CPW // 8)
    def _scat(G):
        for k in range(8):
            pltpu.async_copy(ones_v, acc.at[didx_v.at[G * 8 + k]], dsem,
                             add=True)
        for k in range(8):
            pltpu.make_async_copy(ones_v, acc.at[didx_v.at[G * 8 + k]],
                                  dsem).wait()

    plsc.subcore_barrier()
    pltpu.sync_copy(acc.at[pl.ds(sid * RPS, RPS)],
                    degp_hbm.at[cid, pl.ds(sid * RPS, RPS)])


_deg_pass = functools.partial(
    pl.kernel,
    out_type=jax.ShapeDtypeStruct((NC, NPAD, DW), _f32),
    mesh=_mesh,
    scratch_types=[
        pltpu.VMEM((CPW, CHUNK), jnp.int32),
        pltpu.VMEM((CHUNK, DW), _f32),
        pltpu.VMEM_SHARED((NPAD, DW), _f32),
        pltpu.SemaphoreType.DMA,
    ],
)(_deg_body)


GRP = 40

CPH = E2P // (2 * CHUNK)
CPP = CPH // NW

Q = 4
SUB = CHUNK // Q


def _agg_body(xs_hbm, ids_hbm, agg_hbm,
              sidx_v, didx_v, gbuf0, gbuf1, acc, sem0, sem1):
    cid = lax.axis_index("c")
    sid = lax.axis_index("s")

    @pl.loop(0, CHUNK)
    def _zfill(i):
        for k in range(8):
            gbuf0[i, pl.ds(k * 16, 16)] = jnp.zeros((16,), _f32)

    @pl.loop(0, RPS // CHUNK)
    def _zero(k):
        pltpu.sync_copy(gbuf0, acc.at[pl.ds(sid * RPS + k * CHUNK, CHUNK)])

    plsc.subcore_barrier()

    base_c = (cid * NS + sid) * CPP

    for (s_off, d_off) in ((base_c, CPH + base_c), (CPH + base_c, base_c)):

        @pl.loop(0, CPP // GRP)
        def _grp(g):
            pltpu.sync_copy(ids_hbm.at[pl.ds(s_off + g * GRP, GRP)], sidx_v)
            pltpu.sync_copy(ids_hbm.at[pl.ds(d_off + g * GRP, GRP)], didx_v)

            def _fire(j, buf, sem):
                for q in range(Q):
                    pltpu.async_copy(
                        xs_hbm.at[sidx_v.at[j, pl.ds(q * SUB, SUB)]],
                        buf.at[pl.ds(q * SUB, SUB)], sem)

            def _drain(j, buf, sem):
                for q in range(Q):
                    pltpu.make_async_copy(
                        xs_hbm.at[sidx_v.at[j, pl.ds(q * SUB, SUB)]],
                        buf.at[pl.ds(q * SUB, SUB)], sem).wait()

            _fire(0, gbuf0, sem0)
            _fire(1, gbuf1, sem1)

            @pl.loop(0, GRP // 2)
            def _pair(p):
                j0 = 2 * p
                for (j, buf, sem) in ((j0, gbuf0, sem0), (j0 + 1, gbuf1, sem1)):
                    _drain(j, buf, sem)
                    pltpu.sync_copy(buf, acc.at[didx_v.at[j]], add=True)

                    @pl.when(j + 2 < GRP)
                    def _pre():
                        _fire(j + 2, buf, sem)

    plsc.subcore_barrier()
    pltpu.sync_copy(acc.at[pl.ds(sid * RPS, RPS)],
                    agg_hbm.at[cid, pl.ds(sid * RPS, RPS)])


_agg_pass = functools.partial(
    pl.kernel,
    out_type=jax.ShapeDtypeStruct((NC, NPAD, D), _f32),
    mesh=_mesh,
    scratch_types=[
        pltpu.VMEM((GRP, CHUNK), jnp.int32),
        pltpu.VMEM((GRP, CHUNK), jnp.int32),
        pltpu.VMEM((CHUNK, D), _f32),
        pltpu.VMEM((CHUNK, D), _f32),
        pltpu.VMEM_SHARED((NPAD, D), _f32),
        pltpu.SemaphoreType.DMA,
        pltpu.SemaphoreType.DMA,
    ],
)(_agg_body)


BR = 512


def _tc_a_kernel(x_ref, w_ref, degp_ref, xs_ref, norm_ref):
    deg = degp_ref[0, :, 0:1] + degp_ref[1, :, 0:1] + 1.0
    norm = lax.rsqrt(deg)
    xw = lax.dot_general(x_ref[...], w_ref[...], (((1,), (0,)), ((), ())),
                         precision=lax.Precision.HIGHEST,
                         preferred_element_type=_f32)
    xs_ref[...] = xw * norm
    norm_ref[...] = jnp.broadcast_to(norm, (BR, D))


def _tc_b_kernel(agg_ref, xs1_ref, norm_ref, w_ref, xs2_ref):
    acc = agg_ref[0] + agg_ref[1] + xs1_ref[...]
    h = jnp.maximum(acc * norm_ref[...], 0.0)
    hw = lax.dot_general(h, w_ref[...], (((1,), (0,)), ((), ())),
                         precision=lax.Precision.HIGHEST,
                         preferred_element_type=_f32)
    xs2_ref[...] = hw * norm_ref[...]


def _tc_c_kernel(agg_ref, xs2_ref, norm_ref, o_ref):
    o_ref[...] = (agg_ref[0] + agg_ref[1] + xs2_ref[...]) * norm_ref[...]


def _row_spec(shape3=False):
    if shape3:
        return pl.BlockSpec((NC, BR, D), lambda i: (0, i, 0))
    return pl.BlockSpec((BR, D), lambda i: (i, 0))


_W_SPEC = pl.BlockSpec((D, D), lambda i: (0, 0))
_GRID = (NPAD // BR,)

_tc_a = pl.pallas_call(
    _tc_a_kernel,
    grid=_GRID,
    in_specs=[_row_spec(), _W_SPEC, pl.BlockSpec((NC, BR, DW), lambda i: (0, i, 0))],
    out_specs=[_row_spec(), _row_spec()],
    out_shape=[jax.ShapeDtypeStruct((NPAD, D), _f32),
               jax.ShapeDtypeStruct((NPAD, D), _f32)],
)

_tc_b = pl.pallas_call(
    _tc_b_kernel,
    grid=_GRID,
    in_specs=[_row_spec(True), _row_spec(), _row_spec(), _W_SPEC],
    out_specs=_row_spec(),
    out_shape=jax.ShapeDtypeStruct((NPAD, D), _f32),
)

_tc_c = pl.pallas_call(
    _tc_c_kernel,
    grid=_GRID,
    in_specs=[_row_spec(True), _row_spec(), _row_spec()],
    out_specs=_row_spec(),
    out_shape=jax.ShapeDtypeStruct((NPAD, D), _f32),
)


def kernel(edge_index, emb_weight, W1, W2):
    npad_e = CPH * CHUNK - N_EDGES
    pad = PADROW + (jnp.arange(npad_e, dtype=jnp.int32) % (NPAD - N_NODES))
    ei_pad = jnp.concatenate([edge_index, jnp.stack([pad, pad])], axis=1)
    ids = ei_pad.reshape(2 * CPH, CHUNK)

    x = jnp.pad(emb_weight, ((0, NPAD - N_NODES), (0, 0)))

    degp = _deg_pass(ids)
    xs1, norm = _tc_a(x, W1, degp)
    agg1 = _agg_pass(xs1, ids)
    xs2 = _tc_b(agg1, xs1, norm, W2)
    agg2 = _agg_pass(xs2, ids)
    out = _tc_c(agg2, xs2, norm)
    return out[:N_NODES]
